# baseline (speedup 1.0000x reference)
import functools

import numpy as np
import jax
import jax.numpy as jnp
from jax import lax
from jax.experimental import pallas as pl
from jax.experimental.pallas import tpu as pltpu
from jax.experimental.pallas import tpu_sc as plsc

B = 16384
K = 16
D = 16
NREL = 32
KD = K * D
KR = K * NREL

NW = 32
BPW = B // NW
CHUNK = 128

N2 = (B * K) // NW
SB = 2048


def _f32(x):
    return np.asarray(x, np.float32)


def _group_consts():
    G = np.zeros((KD, K), np.float32)
    T2 = np.zeros((KD, D), np.float32)
    for k in range(K):
        for d in range(D):
            G[k * D + d, k] = 1.0
            T2[k * D + d, d] = 1.0
    G32T = np.zeros((K, KR), np.float32)
    T32 = np.zeros((NREL, KR), np.float32)
    for k in range(K):
        for j in range(NREL):
            G32T[k, k * NREL + j] = 1.0
            T32[j, k * NREL + j] = 1.0
    return G, G.T.copy(), T2, G32T, T32, G32T.T.copy()


_G, _GT, _T2, _G32T, _T32, _R512 = _group_consts()


_SC_PARAMS = pltpu.CompilerParams(use_tc_tiling_on_sc=False)


def _sc_phase1(users, items, adj_entity, adj_relation, user_table, entity_table):
    mesh = plsc.VectorSubcoreMesh(core_axis_name="c", subcore_axis_name="s")
    out_types = (
        jax.ShapeDtypeStruct((B, K), jnp.int32),
        jax.ShapeDtypeStruct((B, K), jnp.int32),
        jax.ShapeDtypeStruct((B, D), jnp.float32),
        jax.ShapeDtypeStruct((B, D), jnp.float32),
    )

    @functools.partial(
        pl.kernel,
        mesh=mesh,
        out_type=out_types,
        scratch_types=[
            pltpu.VMEM((BPW,), jnp.int32),
            pltpu.VMEM((BPW,), jnp.int32),
            pltpu.VMEM((BPW, K), jnp.int32),
            pltpu.VMEM((BPW, K), jnp.int32),
            pltpu.VMEM((BPW, D), jnp.float32),
            pltpu.VMEM((BPW, D), jnp.float32),
            pltpu.SemaphoreType.DMA,
        ],
        compiler_params=_SC_PARAMS,
    )
    def k(users_hbm, items_hbm, adje_hbm, adjr_hbm, ut_hbm, et_hbm,
          nbe_hbm, nbr_hbm, u_hbm, i_hbm,
          uidx_v, iidx_v, nbe_v, nbr_v, u_v, i_v, sem):
        wid = lax.axis_index("s") * 2 + lax.axis_index("c")
        base = pl.multiple_of(wid * BPW, BPW)
        pltpu.sync_copy(users_hbm.at[pl.ds(base, BPW)], uidx_v)
        pltpu.sync_copy(items_hbm.at[pl.ds(base, BPW)], iidx_v)
        copies = []
        for j in range(BPW // CHUNK):
            sl = pl.ds(j * CHUNK, CHUNK)
            copies.append(pltpu.async_copy(adje_hbm.at[iidx_v.at[sl]], nbe_v.at[sl], sem))
            copies.append(pltpu.async_copy(adjr_hbm.at[iidx_v.at[sl]], nbr_v.at[sl], sem))
            copies.append(pltpu.async_copy(ut_hbm.at[uidx_v.at[sl]], u_v.at[sl], sem))
            copies.append(pltpu.async_copy(et_hbm.at[iidx_v.at[sl]], i_v.at[sl], sem))
        for c in copies:
            c.wait()
        pltpu.sync_copy(nbe_v, nbe_hbm.at[pl.ds(base, BPW)])
        pltpu.sync_copy(nbr_v, nbr_hbm.at[pl.ds(base, BPW)])
        pltpu.sync_copy(u_v, u_hbm.at[pl.ds(base, BPW)])
        pltpu.sync_copy(i_v, i_hbm.at[pl.ds(base, BPW)])

    return k(users, items, adj_entity, adj_relation, user_table, entity_table)


def _sc_phase2(ids_flat, entity_table):
    mesh = plsc.VectorSubcoreMesh(core_axis_name="c", subcore_axis_name="s")

    @functools.partial(
        pl.kernel,
        mesh=mesh,
        out_type=jax.ShapeDtypeStruct((B * K, D), jnp.float32),
        scratch_types=[
            pltpu.VMEM((N2,), jnp.int32),
            pltpu.VMEM((SB, D), jnp.float32),
            pltpu.SemaphoreType.DMA,
        ],
        compiler_params=_SC_PARAMS,
    )
    def k(ids_hbm, et_hbm, out_hbm, idx_v, rows_v, sem):
        wid = lax.axis_index("s") * 2 + lax.axis_index("c")
        base = pl.multiple_of(wid * N2, N2)
        pltpu.sync_copy(ids_hbm.at[pl.ds(base, N2)], idx_v)

        @pl.loop(0, N2 // SB)
        def _(sb):
            off = pl.multiple_of(sb * SB, SB)
            copies = []
            for j in range(SB // CHUNK):
                copies.append(pltpu.async_copy(
                    et_hbm.at[idx_v.at[pl.ds(off + j * CHUNK, CHUNK)]],
                    rows_v.at[pl.ds(j * CHUNK, CHUNK)], sem))
            for c in copies:
                c.wait()
            pltpu.sync_copy(rows_v, out_hbm.at[pl.ds(base + off, SB)])

    return k(ids_flat, entity_table)


_EPS = 1e-7
_HI = jax.lax.Precision.HIGHEST


def _dot(a, bm):
    return jax.lax.dot_general(a, bm, (((1,), (0,)), ((), ())),
                               precision=_HI, preferred_element_type=jnp.float32)


def _dot_t(a, bm):
    return jax.lax.dot_general(a, bm, (((1,), (1,)), ((), ())),
                               precision=_HI, preferred_element_type=jnp.float32)


def _renorm_factor(sumsq):
    n = jnp.sqrt(sumsq)
    return jnp.minimum(1.0, 1.0 / jnp.maximum(n, _EPS))


def _tc_body(u_ref, i_ref, ent_ref, rid_ref, rel_ref, w_ref, b_ref,
             g_ref, gt_ref, t2_ref, g32t_ref, t32_ref, r512_ref,
             un_ref, out_ref):
    u = u_ref[...]
    un = u * _renorm_factor(jnp.sum(u * u, axis=1, keepdims=True))
    un_ref[...] = un

    it = i_ref[...]
    inr = it * _renorm_factor(jnp.sum(it * it, axis=1, keepdims=True))

    rel = rel_ref[...]
    relr = rel * _renorm_factor(jnp.sum(rel * rel, axis=1, keepdims=True))
    s_all = _dot_t(un, relr)

    rid = rid_ref[...].astype(jnp.float32)
    rid_t = _dot(rid, g32t_ref[...])
    jj = (lax.broadcasted_iota(jnp.int32, rid_t.shape, 1) % NREL).astype(jnp.float32)
    onehot = (rid_t == jj).astype(jnp.float32)
    s_t = _dot(s_all, t32_ref[...])
    scores = _dot(onehot * s_t, r512_ref[...])

    m = jnp.max(scores, axis=1, keepdims=True)
    e = jnp.exp(scores - m)
    w = e / jnp.sum(e, axis=1, keepdims=True)

    ent = ent_ref[...]
    qe = _dot(ent * ent, g_ref[...])
    fw = _renorm_factor(qe) * w
    fw_t = _dot(fw, gt_ref[...])
    nv = _dot(ent * fw_t, t2_ref[...])

    out = _dot_t(inr + nv, w_ref[...]) + b_ref[...]
    out_ref[...] = jnp.maximum(out, 0.0)


def _tc_attention(u_raw, i_raw, ent_packed, relids, rel_table, W, b,
                  interpret=False):
    BB = 1024
    grid = (B // BB,)

    def row_spec(width):
        return pl.BlockSpec((BB, width), lambda i: (i, 0))

    def full_spec(shape):
        return pl.BlockSpec(shape, lambda i: (0,) * len(shape))

    return pl.pallas_call(
        _tc_body,
        grid=grid,
        in_specs=[
            row_spec(D),
            row_spec(D),
            row_spec(KD),
            row_spec(K),
            full_spec((NREL, D)),
            full_spec((D, D)),
            full_spec((1, D)),
            full_spec((KD, K)),
            full_spec((K, KD)),
            full_spec((KD, D)),
            full_spec((K, KR)),
            full_spec((NREL, KR)),
            full_spec((KR, K)),
        ],
        out_specs=[row_spec(D), row_spec(D)],
        out_shape=[
            jax.ShapeDtypeStruct((B, D), jnp.float32),
            jax.ShapeDtypeStruct((B, D), jnp.float32),
        ],
        interpret=interpret,
    )(u_raw, i_raw, ent_packed, relids, rel_table, W, b.reshape(1, D),
      _G, _GT, _T2, _G32T, _T32, _R512)


def kernel(users, items, adj_entity, adj_relation, user_table, entity_table,
           rel_table, W, b):
    nbe, nbr, u_raw, i_raw = _sc_phase1(
        users, items, adj_entity, adj_relation, user_table, entity_table)
    ent_rows = _sc_phase2(nbe.reshape(B * K), entity_table)
    ent_packed = ent_rows.reshape(B, KD)
    un, out = _tc_attention(u_raw, i_raw, ent_packed, nbr, rel_table, W, b)
    return (un, out)
